# Initial kernel scaffold; baseline (speedup 1.0000x reference)
import jax
import jax.numpy as jnp
from jax.experimental import pallas as pl


def kernel(x, emb, W):
    raise NotImplementedError("write your pallas kernel here")


# speedup vs baseline: 11.5927x; 11.5927x over previous
import functools

import jax
import jax.numpy as jnp
from jax import lax
from jax.experimental import pallas as pl
from jax.experimental.pallas import tpu as pltpu
from jax.experimental.pallas import tpu_sc as plsc

VOCAB = 100000
DIM_WORD = 300
DIM_HIDDEN = 512
B = 4096
L = 200


_BLOCK_M = 4000


def _proj_body(emb_ref, w_ref, out_ref):
    out_ref[...] = lax.dot_general(
        emb_ref[...],
        w_ref[...],
        dimension_numbers=(((1,), (1,)), ((), ())),
        preferred_element_type=jnp.float32,
    )


def _project_table(emb, W):
    return pl.pallas_call(
        _proj_body,
        grid=(VOCAB // _BLOCK_M,),
        in_specs=[
            pl.BlockSpec((_BLOCK_M, DIM_WORD), lambda i: (i, 0)),
            pl.BlockSpec((DIM_HIDDEN, DIM_WORD), lambda i: (0, 0)),
        ],
        out_specs=pl.BlockSpec((_BLOCK_M, DIM_HIDDEN), lambda i: (i, 0)),
        out_shape=jax.ShapeDtypeStruct((VOCAB, DIM_HIDDEN), jnp.float32),
    )(emb, W)


def _make_gather(n_rows, d):
    info = plsc.get_sparse_core_info()
    nw = info.num_cores * info.num_subcores
    chunk = 128
    rows_per_w = n_rows // nw
    n_chunks = rows_per_w // chunk
    mesh = plsc.VectorSubcoreMesh(core_axis_name="c", subcore_axis_name="s")

    @functools.partial(
        pl.kernel,
        mesh=mesh,
        out_type=jax.ShapeDtypeStruct((n_rows, d), jnp.float32),
        scratch_types=[
            pltpu.VMEM((chunk,), jnp.int32),
            pltpu.VMEM((chunk, d), jnp.float32),
            pltpu.SemaphoreType.DMA,
        ],
    )
    def gather(table_hbm, idx_hbm, out_hbm, idx_v, rows_v, sem):
        wid = lax.axis_index("s") * info.num_cores + lax.axis_index("c")
        base0 = wid * rows_per_w

        def body(c, carry):
            base = base0 + c * chunk
            pltpu.sync_copy(idx_hbm.at[pl.ds(base, chunk)], idx_v)
            pltpu.async_copy(table_hbm.at[idx_v], rows_v, sem).wait()
            pltpu.sync_copy(rows_v, out_hbm.at[pl.ds(base, chunk)])
            return carry

        lax.fori_loop(0, n_chunks, body, 0)

    return gather


_gather_rows = _make_gather(B * L, DIM_HIDDEN)


def kernel(x, emb, W):
    table = _project_table(emb, W)
    flat = x.reshape(-1).astype(jnp.int32)
    out = _gather_rows(table, flat)
    return out.reshape(x.shape[0], x.shape[1], DIM_HIDDEN)


# speedup vs baseline: 13.1873x; 1.1376x over previous
import functools

import jax
import jax.numpy as jnp
from jax import lax
from jax.experimental import pallas as pl
from jax.experimental.pallas import tpu as pltpu
from jax.experimental.pallas import tpu_sc as plsc

VOCAB = 100000
DIM_WORD = 300
DIM_HIDDEN = 512
B = 4096
L = 200


_BLOCK_M = 4000


def _proj_body(emb_ref, w_ref, out_ref):
    out_ref[...] = lax.dot_general(
        emb_ref[...],
        w_ref[...],
        dimension_numbers=(((1,), (1,)), ((), ())),
        preferred_element_type=jnp.float32,
    )


def _project_table(emb, W):
    return pl.pallas_call(
        _proj_body,
        grid=(VOCAB // _BLOCK_M,),
        in_specs=[
            pl.BlockSpec((_BLOCK_M, DIM_WORD), lambda i: (i, 0)),
            pl.BlockSpec((DIM_HIDDEN, DIM_WORD), lambda i: (0, 0)),
        ],
        out_specs=pl.BlockSpec((_BLOCK_M, DIM_HIDDEN), lambda i: (i, 0)),
        out_shape=jax.ShapeDtypeStruct((VOCAB, DIM_HIDDEN), jnp.float32),
    )(emb, W)


def _make_gather(n_rows, d, chunk=64, nbuf=2):
    info = plsc.get_sparse_core_info()
    nw = info.num_cores * info.num_subcores
    rows_per_w = n_rows // nw
    n_chunks = rows_per_w // chunk
    mesh = plsc.VectorSubcoreMesh(core_axis_name="c", subcore_axis_name="s")

    scratch = (
        [pltpu.VMEM((rows_per_w,), jnp.int32)]
        + [pltpu.VMEM((chunk, d), jnp.float32) for _ in range(nbuf)]
        + [pltpu.SemaphoreType.DMA for _ in range(2 * nbuf)]
    )

    @functools.partial(
        pl.kernel,
        mesh=mesh,
        out_type=jax.ShapeDtypeStruct((n_rows, d), jnp.float32),
        scratch_types=scratch,
    )
    def gather(table_hbm, idx_hbm, out_hbm, idx_all, *bufs_sems):
        rows = bufs_sems[:nbuf]
        gsem = bufs_sems[nbuf : 2 * nbuf]
        ssem = bufs_sems[2 * nbuf :]
        wid = lax.axis_index("s") * info.num_cores + lax.axis_index("c")
        base0 = wid * rows_per_w

        pltpu.sync_copy(idx_hbm.at[pl.ds(base0, rows_per_w)], idx_all)

        def g_copy(b, c):
            return pltpu.make_async_copy(
                table_hbm.at[idx_all.at[pl.ds(c * chunk, chunk)]],
                rows[b],
                gsem[b],
            )

        def s_copy(b, c):
            return pltpu.make_async_copy(
                rows[b],
                out_hbm.at[pl.ds(base0 + c * chunk, chunk)],
                ssem[b],
            )

        g_copy(0, 0).start()

        def body(i, carry):
            b = lax.rem(i, nbuf)

            def per_buf(bb):
                @pl.when(b == bb)
                def _():
                    g_copy(bb, i).wait()
                    s_copy(bb, i).start()

                b2 = (bb + 1) % nbuf

                @pl.when((b == bb) & (i + 1 >= nbuf))
                def _():
                    s_copy(b2, i + 1 - nbuf).wait()

                @pl.when((b == bb) & (i + 1 < n_chunks))
                def _():
                    g_copy(b2, i + 1).start()

            for bb in range(nbuf):
                per_buf(bb)
            return carry

        lax.fori_loop(0, n_chunks, body, 0)
        for c in range(n_chunks - nbuf + 1, n_chunks):
            s_copy(c % nbuf, c).wait()

    return gather


_gather_rows = _make_gather(B * L, DIM_HIDDEN)


def kernel(x, emb, W):
    table = _project_table(emb, W)
    flat = x.reshape(-1).astype(jnp.int32)
    out = _gather_rows(table, flat)
    return out.reshape(x.shape[0], x.shape[1], DIM_HIDDEN)
